# baseline (speedup 1.0000x reference)
import jax
import jax.numpy as jnp
import numpy as np
from jax.experimental import pallas as pl
from jax.experimental.pallas import tpu as pltpu

B = 1024
D = 512
H = 8
DH = D // H
BF = jnp.bfloat16
HPS = 4


def _mm_t(a, w):
    return jax.lax.dot_general(a, w, (((1,), (1,)), ((), ())),
                               preferred_element_type=jnp.float32)


def _fused_kernel(x_ref, W1_ref, b1_ref, W2_ref, b2_ref,
                  Wq_ref, bq_ref, Wk_ref, Wv_ref, bv_ref,
                  Wo_ref, bo_ref, W3_ref, b3_ref,
                  W4_ref, b4_ref, out_ref,
                  q_scr, k_scr, v_scr, att_scr, pat_scr):
    g = pl.program_id(0)

    @pl.when(g == 0)
    def _proj():
        x = x_ref[...].astype(BF)
        h = jnp.maximum(_mm_t(x, W1_ref[...].astype(BF)) + b1_ref[...],
                        0.0).astype(BF)
        pat_scr[...] = jnp.maximum(
            _mm_t(h, W2_ref[...].astype(BF)) + b2_ref[...], 0.0).astype(BF)
        scale = np.float32(1.0 / np.sqrt(DH))
        q_scr[...] = ((_mm_t(x, Wq_ref[...].astype(BF)) + bq_ref[...])
                      * scale).astype(BF)
        k_scr[...] = _mm_t(x, Wk_ref[...].astype(BF)).astype(BF)
        v_scr[...] = (_mm_t(x, Wv_ref[...].astype(BF)) + bv_ref[...]).astype(BF)

    half = HPS * DH
    off = g * half
    qg = q_scr[:, pl.ds(off, half)]
    kg = k_scr[:, pl.ds(off, half)]
    vg = v_scr[:, pl.ds(off, half)]

    col = jax.lax.broadcasted_iota(jnp.int32, (B, DH), 1)
    ones_blk = (col == 0).astype(BF)

    head_outs = []
    for hi in range(HPS):
        qh = qg[:, hi * DH:(hi + 1) * DH]
        kh = kg[:, hi * DH:(hi + 1) * DH]
        vh = jnp.concatenate([vg[:, hi * DH:(hi + 1) * DH], ones_blk], axis=-1)
        s = jax.lax.dot_general(qh, kh, (((1,), (1,)), ((), ())),
                                preferred_element_type=jnp.float32).astype(BF)
        m = jnp.max(s, axis=-1, keepdims=True)
        e = jnp.exp(s - m)
        o2 = jnp.dot(e, vh, preferred_element_type=jnp.float32)
        r = 1.0 / o2[:, DH:DH + 1]
        head_outs.append((o2[:, :DH] * r).astype(BF))
    att_scr[:, pl.ds(off, half)] = jnp.concatenate(head_outs, axis=-1)

    @pl.when(g == 1)
    def _finish():
        att = att_scr[...]
        attended = (_mm_t(att, Wo_ref[...].astype(BF)) + bo_ref[...]).astype(BF)
        W3 = W3_ref[...].astype(BF)
        h2 = jnp.maximum(_mm_t(pat_scr[...], W3[:, :128])
                         + _mm_t(attended, W3[:, 128:]) + b3_ref[...], 0.0)
        out_ref[...] = jnp.tanh(_mm_t(h2.astype(BF), W4_ref[...].astype(BF))
                                + b4_ref[...])


def kernel(sensory_input, W1, b1, W2, b2, Wq, bq, Wk, bk, Wv, bv, Wo, bo, W3, b3, W4, b4):
    del bk
    full = lambda shape: pl.BlockSpec(shape, lambda i: tuple(0 for _ in shape))
    return pl.pallas_call(
        _fused_kernel,
        grid=(2,),
        in_specs=[
            full((B, D)),
            full((256, D)), full((256,)),
            full((128, 256)), full((128,)),
            full((D, D)), full((D,)),
            full((D, D)),
            full((D, D)), full((D,)),
            full((D, D)), full((D,)),
            full((256, 128 + D)), full((256,)),
            full((D, 256)), full((D,)),
        ],
        out_specs=full((B, D)),
        out_shape=jax.ShapeDtypeStruct((B, D), jnp.float32),
        scratch_shapes=[pltpu.VMEM((B, D), BF), pltpu.VMEM((B, D), BF),
                        pltpu.VMEM((B, D), BF), pltpu.VMEM((B, D), BF),
                        pltpu.VMEM((B, 128), BF)],
    )(sensory_input, W1, b1, W2, b2, Wq, bq, Wk, Wv, bv, Wo, bo, W3, b3, W4, b4)


# speedup vs baseline: 1.1272x; 1.1272x over previous
import jax
import jax.numpy as jnp
import numpy as np
from jax.experimental import pallas as pl

B = 1024
D = 512
H = 8
DH = D // H
BF = jnp.bfloat16


def _mm_t(a, w):
    return jax.lax.dot_general(a, w, (((1,), (1,)), ((), ())),
                               preferred_element_type=jnp.float32)


def _fused_kernel(x_ref, W1_ref, b1_ref, W2_ref, b2_ref,
                  Wq_ref, bq_ref, Wk_ref, Wv_ref, bv_ref,
                  Wo_ref, bo_ref, W3_ref, b3_ref,
                  W4_ref, b4_ref, out_ref):
    x = x_ref[...].astype(BF)
    h = jnp.maximum(_mm_t(x, W1_ref[...].astype(BF)) + b1_ref[...], 0.0).astype(BF)
    patterns = jnp.maximum(_mm_t(h, W2_ref[...].astype(BF)) + b2_ref[...],
                           0.0).astype(BF)

    scale = np.float32(1.0 / np.sqrt(DH))
    q = ((_mm_t(x, Wq_ref[...].astype(BF)) + bq_ref[...]) * scale).astype(BF)
    k = _mm_t(x, Wk_ref[...].astype(BF)).astype(BF)
    v = (_mm_t(x, Wv_ref[...].astype(BF)) + bv_ref[...]).astype(BF)

    col = jax.lax.broadcasted_iota(jnp.int32, (B, DH), 1)
    ones_blk = (col == 0).astype(BF)

    head_outs = []
    for hh in range(H):
        qh = q[:, hh * DH:(hh + 1) * DH]
        kh = k[:, hh * DH:(hh + 1) * DH]
        vh = jnp.concatenate([v[:, hh * DH:(hh + 1) * DH], ones_blk], axis=-1)
        s = jax.lax.dot_general(qh, kh, (((1,), (1,)), ((), ())),
                                preferred_element_type=jnp.float32).astype(BF)
        m = jnp.max(s, axis=-1, keepdims=True)
        e = jnp.exp(s - m)
        o2 = jnp.dot(e, vh, preferred_element_type=jnp.float32)
        r = 1.0 / o2[:, DH:DH + 1]
        head_outs.append((o2[:, :DH] * r).astype(BF))
    att = jnp.concatenate(head_outs, axis=-1)
    attended = (_mm_t(att, Wo_ref[...].astype(BF)) + bo_ref[...]).astype(BF)

    W3 = W3_ref[...].astype(BF)
    h2 = jnp.maximum(_mm_t(patterns, W3[:, :128])
                     + _mm_t(attended, W3[:, 128:]) + b3_ref[...], 0.0)
    out_ref[...] = jnp.tanh(_mm_t(h2.astype(BF), W4_ref[...].astype(BF))
                            + b4_ref[...])


def kernel(sensory_input, W1, b1, W2, b2, Wq, bq, Wk, bk, Wv, bv, Wo, bo, W3, b3, W4, b4):
    del bk
    return pl.pallas_call(
        _fused_kernel,
        out_shape=jax.ShapeDtypeStruct((B, D), jnp.float32),
    )(sensory_input, W1, b1, W2, b2, Wq, bq, Wk, Wv, bv, Wo, bo, W3, b3, W4, b4)
